# Initial kernel scaffold; baseline (speedup 1.0000x reference)
import jax
import jax.numpy as jnp
from jax.experimental import pallas as pl


def kernel(inputs, gamma, beta, moving_mean, moving_var, sp_values, sp_rows, sp_cols):
    raise NotImplementedError("write your pallas kernel here")


# speedup vs baseline: 1.8877x; 1.8877x over previous
import functools
import jax
import jax.numpy as jnp
from jax import lax
from jax.experimental import pallas as pl
from jax.experimental.pallas import tpu as pltpu
from jax.experimental.pallas import tpu_sc as plsc

_NUM_IN = 16384
_NUM_OUT = 16384
_BATCH = 256
_NNZ = 268435
_EPS = 1e-3

_NC = 2
_NS = 16
_NW = _NC * _NS
_L = 16

_K = 8192
_NCHUNK = 33
_NNZ_PAD = _K * _NCHUNK
_GROUPS = _K // _L
_ROWS_PER_W = _BATCH // _NW
_RES = 2
_PASSES = _ROWS_PER_W // _RES


def _bn_body(x_ref, s_ref, b_ref, o_ref):
    o_ref[...] = x_ref[...] * s_ref[...][None, :] + b_ref[...][None, :]


def _bn_tc(x, scale, bias):
    nblk = 16
    blk = _NUM_IN // nblk
    return pl.pallas_call(
        _bn_body,
        out_shape=jax.ShapeDtypeStruct((_BATCH, _NUM_IN), jnp.float32),
        grid=(nblk,),
        in_specs=[
            pl.BlockSpec((_BATCH, blk), lambda i: (0, i)),
            pl.BlockSpec((blk,), lambda i: (i,)),
            pl.BlockSpec((blk,), lambda i: (i,)),
        ],
        out_specs=pl.BlockSpec((_BATCH, blk), lambda i: (0, i)),
    )(x, scale, bias)


def _sc_body(h_hbm, rows_hbm, cols_hbm, vals_hbm, out_hbm,
             hb, accb, rowsb, colsb, valsb):
    wid = lax.axis_index("s") * _NC + lax.axis_index("c")

    zero16 = jnp.zeros((_L,), jnp.float32)

    for p in range(_PASSES):
        b0 = wid * _ROWS_PER_W + p * _RES
        for i in range(_RES):
            pltpu.sync_copy(h_hbm.at[b0 + i], hb.at[pl.ds(i * _NUM_IN, _NUM_IN)])

        def _zero(j, _):
            accb[pl.ds(j * _L, _L)] = zero16
            return 0
        lax.fori_loop(0, _RES * _NUM_OUT // _L, _zero, 0)

        def _chunk(c, _):
            pltpu.sync_copy(rows_hbm.at[pl.ds(c * _K, _K)], rowsb)
            pltpu.sync_copy(cols_hbm.at[pl.ds(c * _K, _K)], colsb)
            pltpu.sync_copy(vals_hbm.at[pl.ds(c * _K, _K)], valsb)

            def _group(g, _):
                rv = rowsb[pl.ds(g * _L, _L)]
                cv = colsb[pl.ds(g * _L, _L)]
                vv = valsb[pl.ds(g * _L, _L)]
                for i in range(_RES):
                    gat = plsc.load_gather(hb, [rv + (i * _NUM_IN)])
                    plsc.addupdate_scatter(accb, [cv + (i * _NUM_OUT)], gat * vv)
                return 0
            lax.fori_loop(0, _GROUPS, _group, 0)
            return 0
        lax.fori_loop(0, _NCHUNK, _chunk, 0)

        for i in range(_RES):
            pltpu.sync_copy(accb.at[pl.ds(i * _NUM_OUT, _NUM_OUT)], out_hbm.at[b0 + i])


def _sc_sparse_matmul(h, rows, cols, vals):
    mesh = plsc.VectorSubcoreMesh(core_axis_name="c", subcore_axis_name="s")
    f = pl.kernel(
        _sc_body,
        out_type=jax.ShapeDtypeStruct((_BATCH, _NUM_OUT), jnp.float32),
        mesh=mesh,
        compiler_params=pltpu.CompilerParams(needs_layout_passes=False),
        scratch_types=[
            pltpu.VMEM((_RES * _NUM_IN,), jnp.float32),
            pltpu.VMEM((_RES * _NUM_OUT,), jnp.float32),
            pltpu.VMEM((_K,), jnp.int32),
            pltpu.VMEM((_K,), jnp.int32),
            pltpu.VMEM((_K,), jnp.float32),
        ],
    )
    return f(h, rows, cols, vals)


def kernel(inputs, gamma, beta, moving_mean, moving_var,
           sp_values, sp_rows, sp_cols):
    scale = gamma * lax.rsqrt(moving_var + _EPS)
    bias = beta - moving_mean * scale

    pad = _NNZ_PAD - _NNZ
    rows = jnp.concatenate([sp_rows, jnp.zeros((pad,), jnp.int32)])
    cols = jnp.concatenate([sp_cols, jnp.zeros((pad,), jnp.int32)])
    vals = jnp.concatenate([sp_values, jnp.zeros((pad,), jnp.float32)])

    h = _bn_tc(inputs, scale, bias)
    return _sc_sparse_matmul(h, rows, cols, vals)
